# baseline (speedup 1.0000x reference)
import functools

import jax
import jax.numpy as jnp
from jax import lax
from jax.experimental import pallas as pl
from jax.experimental.pallas import tpu as pltpu
from jax.experimental.pallas import tpu_sc as plsc

N = 10000
NP = 10240
E = 320000
NW = 32
KCH = 80
EW = KCH * 128
EP = EW * NW
RB = 512
GRID = NP // RB
EPS = 1e-5

_mesh = plsc.VectorSubcoreMesh(core_axis_name="c", subcore_axis_name="s")


@functools.partial(
    pl.kernel,
    mesh=_mesh,
    out_type=jax.ShapeDtypeStruct((2, NP, 128), jnp.float32),
    scratch_types=[
        pltpu.VMEM((KCH, 128), jnp.int32),
        pltpu.VMEM((128, 128), jnp.float32),
        pltpu.VMEM_SHARED((NP, 128), jnp.float32),
    ],
)
def _sc_degree(dst_hbm, zeros_hbm, oh_hbm, out_hbm, dst_v, rows_v, acc_sh):
    cid = lax.axis_index("c")
    sid = lax.axis_index("s")
    wid = sid * 2 + cid
    pltpu.sync_copy(zeros_hbm.at[pl.ds(sid * 640, 640)],
                    acc_sh.at[pl.ds(sid * 640, 640)])
    pltpu.sync_copy(dst_hbm.at[wid], dst_v)
    pltpu.sync_copy(oh_hbm, rows_v)
    plsc.subcore_barrier()

    def body(j, carry):
        pltpu.sync_copy(rows_v, acc_sh.at[dst_v.at[j]], add=True)
        return carry

    lax.fori_loop(0, KCH, body, 0)
    plsc.subcore_barrier()
    pltpu.sync_copy(acc_sh.at[pl.ds(sid * 640, 640)],
                    out_hbm.at[cid, pl.ds(sid * 640, 640)])


def _make_sc_scatter(F):
    @functools.partial(
        pl.kernel,
        mesh=_mesh,
        out_type=jax.ShapeDtypeStruct((2, NP, F), jnp.float32),
        scratch_types=[
            pltpu.VMEM((KCH, 128), jnp.int32),
            pltpu.VMEM((KCH, 128), jnp.int32),
            pltpu.VMEM((128, F), jnp.float32),
            pltpu.VMEM_SHARED((NP, F), jnp.float32),
            pltpu.SemaphoreType.DMA,
        ],
    )
    def sc_scatter(u_hbm, src_hbm, dst_hbm, zeros_hbm, out_hbm,
                   src_v, dst_v, rows_v, acc_sh, sem):
        cid = lax.axis_index("c")
        sid = lax.axis_index("s")
        wid = sid * 2 + cid
        pltpu.sync_copy(zeros_hbm.at[pl.ds(sid * 640, 640)],
                        acc_sh.at[pl.ds(sid * 640, 640)])
        pltpu.sync_copy(src_hbm.at[wid], src_v)
        pltpu.sync_copy(dst_hbm.at[wid], dst_v)
        plsc.subcore_barrier()

        def body(j, carry):
            pltpu.async_copy(u_hbm.at[src_v.at[j]], rows_v, sem).wait()
            pltpu.sync_copy(rows_v, acc_sh.at[dst_v.at[j]], add=True)
            return carry

        lax.fori_loop(0, KCH, body, 0)
        plsc.subcore_barrier()
        pltpu.sync_copy(acc_sh.at[pl.ds(sid * 640, 640)],
                        out_hbm.at[cid, pl.ds(sid * 640, 640)])

    return sc_scatter


_sc_scatter_128 = _make_sc_scatter(128)


def _dinv_scale_body(parts_ref, x_ref, dinv_ref, u_ref):
    s = jnp.sum(parts_ref[...], axis=(0, 2)) * (1.0 / 128.0)
    dv = lax.rsqrt(1.0 + s)[:, None]
    dinv_ref[...] = dv
    u_ref[...] = x_ref[...] * dv


def _tc_dinv_scale(parts, x):
    return pl.pallas_call(
        _dinv_scale_body,
        grid=(GRID,),
        in_specs=[
            pl.BlockSpec((2, RB, 128), lambda i: (0, i, 0)),
            pl.BlockSpec((RB, 128), lambda i: (i, 0)),
        ],
        out_specs=[
            pl.BlockSpec((RB, 1), lambda i: (i, 0)),
            pl.BlockSpec((RB, 128), lambda i: (i, 0)),
        ],
        out_shape=[
            jax.ShapeDtypeStruct((NP, 1), jnp.float32),
            jax.ShapeDtypeStruct((NP, 128), jnp.float32),
        ],
    )(parts, x)


def _stats(i, z, st_ref):
    rows = i * RB + lax.broadcasted_iota(jnp.int32, (RB, 1), 0)
    zm = jnp.where(rows < N, z, 0.0)
    st = jnp.concatenate(
        [jnp.sum(zm, axis=0, keepdims=True),
         jnp.sum(zm * zm, axis=0, keepdims=True)], axis=0)

    @pl.when(i == 0)
    def _():
        st_ref[...] = st

    @pl.when(i > 0)
    def _():
        st_ref[...] += st


def _mm1_body(agg_ref, u_ref, dinv_ref, w_ref, b_ref, z_ref, st_ref):
    i = pl.program_id(0)
    t = (agg_ref[0] + agg_ref[1] + u_ref[...]) * dinv_ref[...]
    z = jnp.dot(t, w_ref[...], preferred_element_type=jnp.float32) + b_ref[...]
    z_ref[...] = z
    _stats(i, z, st_ref)


def _tc_mm1(agg, u, dinv, w, b):
    Fi, Fo = w.shape
    return pl.pallas_call(
        _mm1_body,
        grid=(GRID,),
        in_specs=[
            pl.BlockSpec((2, RB, Fi), lambda i: (0, i, 0)),
            pl.BlockSpec((RB, Fi), lambda i: (i, 0)),
            pl.BlockSpec((RB, 1), lambda i: (i, 0)),
            pl.BlockSpec((Fi, Fo), lambda i: (0, 0)),
            pl.BlockSpec((1, Fo), lambda i: (0, 0)),
        ],
        out_specs=[
            pl.BlockSpec((RB, Fo), lambda i: (i, 0)),
            pl.BlockSpec((2, Fo), lambda i: (0, 0)),
        ],
        out_shape=[
            jax.ShapeDtypeStruct((NP, Fo), jnp.float32),
            jax.ShapeDtypeStruct((2, Fo), jnp.float32),
        ],
    )(agg, u, dinv, w, b)


def _bnmm_body(z_ref, st_ref, g_ref, be_ref, dinv_ref, w_ref, u_ref):
    st = st_ref[...]
    mean = st[0:1, :] * (1.0 / N)
    var = st[1:2, :] * (1.0 / N) - mean * mean
    scale = g_ref[...] * lax.rsqrt(var + EPS)
    h = jnp.maximum((z_ref[...] - mean) * scale + be_ref[...], 0.0)
    u_ref[...] = jnp.dot(h, w_ref[...],
                         preferred_element_type=jnp.float32) * dinv_ref[...]


def _tc_bnmm(z, st, g, be, dinv, w):
    Fi, Fo = w.shape
    return pl.pallas_call(
        _bnmm_body,
        grid=(GRID,),
        in_specs=[
            pl.BlockSpec((RB, Fi), lambda i: (i, 0)),
            pl.BlockSpec((2, Fi), lambda i: (0, 0)),
            pl.BlockSpec((1, Fi), lambda i: (0, 0)),
            pl.BlockSpec((1, Fi), lambda i: (0, 0)),
            pl.BlockSpec((RB, 1), lambda i: (i, 0)),
            pl.BlockSpec((Fi, Fo), lambda i: (0, 0)),
        ],
        out_specs=pl.BlockSpec((RB, Fo), lambda i: (i, 0)),
        out_shape=jax.ShapeDtypeStruct((NP, Fo), jnp.float32),
    )(z, st, g, be, dinv, w)


def _fin_body(agg_ref, u_ref, dinv_ref, b_ref, z_ref, st_ref):
    i = pl.program_id(0)
    z = (agg_ref[0] + agg_ref[1] + u_ref[...]) * dinv_ref[...] + b_ref[...]
    z_ref[...] = z
    _stats(i, z, st_ref)


def _tc_fin(agg, u, dinv, b):
    F = u.shape[1]
    return pl.pallas_call(
        _fin_body,
        grid=(GRID,),
        in_specs=[
            pl.BlockSpec((2, RB, F), lambda i: (0, i, 0)),
            pl.BlockSpec((RB, F), lambda i: (i, 0)),
            pl.BlockSpec((RB, 1), lambda i: (i, 0)),
            pl.BlockSpec((1, F), lambda i: (0, 0)),
        ],
        out_specs=[
            pl.BlockSpec((RB, F), lambda i: (i, 0)),
            pl.BlockSpec((2, F), lambda i: (0, 0)),
        ],
        out_shape=[
            jax.ShapeDtypeStruct((NP, F), jnp.float32),
            jax.ShapeDtypeStruct((2, F), jnp.float32),
        ],
    )(agg, u, dinv, b)


def _out_body(agg_ref, u_ref, dinv_ref, b_ref, o_ref):
    z = (agg_ref[0] + agg_ref[1] + u_ref[...]) * dinv_ref[...] + b_ref[...]
    col = lax.broadcasted_iota(jnp.int32, (RB, 128), 1)
    zm = jnp.where(col < 40, z, -1e30)
    m = jnp.max(zm, axis=1, keepdims=True)
    s = jnp.sum(jnp.exp(zm - m), axis=1, keepdims=True)
    o_ref[...] = z - m - jnp.log(s)


def _tc_out(agg, u, dinv, b):
    return pl.pallas_call(
        _out_body,
        grid=(GRID,),
        in_specs=[
            pl.BlockSpec((2, RB, 128), lambda i: (0, i, 0)),
            pl.BlockSpec((RB, 128), lambda i: (i, 0)),
            pl.BlockSpec((RB, 1), lambda i: (i, 0)),
            pl.BlockSpec((1, 128), lambda i: (0, 0)),
        ],
        out_specs=pl.BlockSpec((RB, 128), lambda i: (i, 0)),
        out_shape=jax.ShapeDtypeStruct((NP, 128), jnp.float32),
    )(agg, u, dinv, b)


def kernel(x, edge_index, W1, b1, W2, b2, W3, b3, W4, b4,
           g1, be1, g2, be2, g3, be3):
    src = edge_index[0].astype(jnp.int32)
    dst = edge_index[1].astype(jnp.int32)
    pad = jnp.full((EP - E,), N, jnp.int32)
    src_p = jnp.concatenate([src, pad]).reshape(NW, KCH, 128)
    dst_p = jnp.concatenate([dst, pad]).reshape(NW, KCH, 128)

    x_p = jnp.pad(x, ((0, NP - N), (0, 0)))
    W3p = jnp.pad(W3, ((0, 0), (0, 64)))
    b3p = jnp.pad(b3, (0, 64))
    g3p = jnp.pad(g3, (0, 64))
    be3p = jnp.pad(be3, (0, 64))
    W4p = jnp.pad(W4, ((0, 64), (0, 88)))
    b4p = jnp.pad(b4, (0, 88))

    z128 = jnp.zeros((NP, 128), jnp.float32)

    oh = jnp.ones((128, 128), jnp.float32)
    deg_parts = _sc_degree(dst_p, z128, oh)

    b1r, b2r, b3r, b4r = (v.reshape(1, -1) for v in (b1, b2, b3p, b4p))
    g1r, g2r, g3r = (v.reshape(1, -1) for v in (g1, g2, g3p))
    be1r, be2r, be3r = (v.reshape(1, -1) for v in (be1, be2, be3p))

    dinv, u0 = _tc_dinv_scale(deg_parts, x_p)
    agg1 = _sc_scatter_128(u0, src_p, dst_p, z128)
    z1, st1 = _tc_mm1(agg1, u0, dinv, W1, b1r)

    u2 = _tc_bnmm(z1, st1, g1r, be1r, dinv, W2)
    agg2 = _sc_scatter_128(u2, src_p, dst_p, z128)
    z2, st2 = _tc_fin(agg2, u2, dinv, b2r)

    u3 = _tc_bnmm(z2, st2, g2r, be2r, dinv, W3p)
    agg3 = _sc_scatter_128(u3, src_p, dst_p, z128)
    z3, st3 = _tc_fin(agg3, u3, dinv, b3r)

    u4 = _tc_bnmm(z3, st3, g3r, be3r, dinv, W4p)
    agg4 = _sc_scatter_128(u4, src_p, dst_p, z128)
    out = _tc_out(agg4, u4, dinv, b4r)

    return out[:N, :40]


# speedup vs baseline: 2.5296x; 2.5296x over previous
import functools

import jax
import jax.numpy as jnp
from jax import lax
from jax.experimental import pallas as pl
from jax.experimental.pallas import tpu as pltpu
from jax.experimental.pallas import tpu_sc as plsc

N = 10000
NP = 10240
E = 320000
NW = 32
KCH = 80
EW = KCH * 128
EP = EW * NW
RB = 512
GRID = NP // RB
EPS = 1e-5

_mesh = plsc.VectorSubcoreMesh(core_axis_name="c", subcore_axis_name="s")


@functools.partial(
    pl.kernel,
    mesh=_mesh,
    out_type=jax.ShapeDtypeStruct((2, NP, 128), jnp.float32),
    scratch_types=[
        pltpu.VMEM((KCH, 128), jnp.int32),
        pltpu.VMEM((128, 128), jnp.float32),
        pltpu.VMEM_SHARED((NP, 128), jnp.float32),
    ],
)
def _sc_degree(dst_hbm, zeros_hbm, oh_hbm, out_hbm, dst_v, rows_v, acc_sh):
    cid = lax.axis_index("c")
    sid = lax.axis_index("s")
    wid = sid * 2 + cid
    pltpu.sync_copy(zeros_hbm.at[pl.ds(sid * 640, 640)],
                    acc_sh.at[pl.ds(sid * 640, 640)])
    pltpu.sync_copy(dst_hbm.at[wid], dst_v)
    pltpu.sync_copy(oh_hbm, rows_v)
    plsc.subcore_barrier()

    def body(j, carry):
        pltpu.sync_copy(rows_v, acc_sh.at[dst_v.at[j]], add=True)
        return carry

    lax.fori_loop(0, KCH, body, 0)
    plsc.subcore_barrier()
    pltpu.sync_copy(acc_sh.at[pl.ds(sid * 640, 640)],
                    out_hbm.at[cid, pl.ds(sid * 640, 640)])


def _make_sc_scatter(F):
    @functools.partial(
        pl.kernel,
        mesh=_mesh,
        out_type=jax.ShapeDtypeStruct((2, NP, F), jnp.float32),
        scratch_types=[
            pltpu.VMEM((KCH, 128), jnp.int32),
            pltpu.VMEM((KCH, 128), jnp.int32),
            pltpu.VMEM((128, F), jnp.float32),
            pltpu.VMEM_SHARED((NP, F), jnp.float32),
            pltpu.SemaphoreType.DMA,
        ],
    )
    def sc_scatter(u_hbm, src_hbm, dst_hbm, zeros_hbm, out_hbm,
                   src_v, dst_v, rows_v, acc_sh, sem):
        cid = lax.axis_index("c")
        sid = lax.axis_index("s")
        wid = sid * 2 + cid
        pltpu.sync_copy(zeros_hbm.at[pl.ds(sid * 640, 640)],
                        acc_sh.at[pl.ds(sid * 640, 640)])
        pltpu.sync_copy(src_hbm.at[wid], src_v)
        pltpu.sync_copy(dst_hbm.at[wid], dst_v)
        plsc.subcore_barrier()

        def body(j, carry):
            pltpu.async_copy(u_hbm.at[src_v.at[j]], rows_v, sem).wait()
            pltpu.sync_copy(rows_v, acc_sh.at[dst_v.at[j]], add=True)
            return carry

        lax.fori_loop(0, KCH, body, 0)
        plsc.subcore_barrier()
        pltpu.sync_copy(acc_sh.at[pl.ds(sid * 640, 640)],
                        out_hbm.at[cid, pl.ds(sid * 640, 640)])

    return sc_scatter


_sc_scatter_128 = _make_sc_scatter(128)


def _dinv_scale_body(parts_ref, x_ref, dinv_ref, u_ref):
    s = jnp.sum(parts_ref[...], axis=(0, 2)) * (1.0 / 128.0)
    dv = lax.rsqrt(1.0 + s)[:, None]
    dinv_ref[...] = dv
    u_ref[...] = x_ref[...] * dv


def _tc_dinv_scale(parts, x):
    return pl.pallas_call(
        _dinv_scale_body,
        grid=(GRID,),
        in_specs=[
            pl.BlockSpec((2, RB, 128), lambda i: (0, i, 0)),
            pl.BlockSpec((RB, 128), lambda i: (i, 0)),
        ],
        out_specs=[
            pl.BlockSpec((RB, 1), lambda i: (i, 0)),
            pl.BlockSpec((RB, 128), lambda i: (i, 0)),
        ],
        out_shape=[
            jax.ShapeDtypeStruct((NP, 1), jnp.float32),
            jax.ShapeDtypeStruct((NP, 128), jnp.float32),
        ],
    )(parts, x)


def _stats(i, z, st_ref):
    rows = i * RB + lax.broadcasted_iota(jnp.int32, (RB, 1), 0)
    zm = jnp.where(rows < N, z, 0.0)
    st = jnp.concatenate(
        [jnp.sum(zm, axis=0, keepdims=True),
         jnp.sum(zm * zm, axis=0, keepdims=True)], axis=0)

    @pl.when(i == 0)
    def _():
        st_ref[...] = st

    @pl.when(i > 0)
    def _():
        st_ref[...] += st


def _mm1_body(agg_ref, u_ref, dinv_ref, w_ref, b_ref, z_ref, st_ref):
    i = pl.program_id(0)
    t = (agg_ref[0] + agg_ref[1] + u_ref[...]) * dinv_ref[...]
    z = jnp.dot(t, w_ref[...], preferred_element_type=jnp.float32) + b_ref[...]
    z_ref[...] = z
    _stats(i, z, st_ref)


def _tc_mm1(agg, u, dinv, w, b):
    Fi, Fo = w.shape
    return pl.pallas_call(
        _mm1_body,
        grid=(GRID,),
        in_specs=[
            pl.BlockSpec((2, RB, Fi), lambda i: (0, i, 0)),
            pl.BlockSpec((RB, Fi), lambda i: (i, 0)),
            pl.BlockSpec((RB, 1), lambda i: (i, 0)),
            pl.BlockSpec((Fi, Fo), lambda i: (0, 0)),
            pl.BlockSpec((1, Fo), lambda i: (0, 0)),
        ],
        out_specs=[
            pl.BlockSpec((RB, Fo), lambda i: (i, 0)),
            pl.BlockSpec((2, Fo), lambda i: (0, 0)),
        ],
        out_shape=[
            jax.ShapeDtypeStruct((NP, Fo), jnp.float32),
            jax.ShapeDtypeStruct((2, Fo), jnp.float32),
        ],
    )(agg, u, dinv, w, b)


def _bnmm_body(z_ref, st_ref, g_ref, be_ref, dinv_ref, w_ref, u_ref):
    st = st_ref[...]
    mean = st[0:1, :] * (1.0 / N)
    var = st[1:2, :] * (1.0 / N) - mean * mean
    scale = g_ref[...] * lax.rsqrt(var + EPS)
    h = jnp.maximum((z_ref[...] - mean) * scale + be_ref[...], 0.0)
    u_ref[...] = jnp.dot(h, w_ref[...],
                         preferred_element_type=jnp.float32) * dinv_ref[...]


def _tc_bnmm(z, st, g, be, dinv, w):
    Fi, Fo = w.shape
    return pl.pallas_call(
        _bnmm_body,
        grid=(GRID,),
        in_specs=[
            pl.BlockSpec((RB, Fi), lambda i: (i, 0)),
            pl.BlockSpec((2, Fi), lambda i: (0, 0)),
            pl.BlockSpec((1, Fi), lambda i: (0, 0)),
            pl.BlockSpec((1, Fi), lambda i: (0, 0)),
            pl.BlockSpec((RB, 1), lambda i: (i, 0)),
            pl.BlockSpec((Fi, Fo), lambda i: (0, 0)),
        ],
        out_specs=pl.BlockSpec((RB, Fo), lambda i: (i, 0)),
        out_shape=jax.ShapeDtypeStruct((NP, Fo), jnp.float32),
    )(z, st, g, be, dinv, w)


def _fin_body(agg_ref, u_ref, dinv_ref, b_ref, z_ref, st_ref):
    i = pl.program_id(0)
    z = (agg_ref[0] + agg_ref[1] + u_ref[...]) * dinv_ref[...] + b_ref[...]
    z_ref[...] = z
    _stats(i, z, st_ref)


def _tc_fin(agg, u, dinv, b):
    F = u.shape[1]
    return pl.pallas_call(
        _fin_body,
        grid=(GRID,),
        in_specs=[
            pl.BlockSpec((2, RB, F), lambda i: (0, i, 0)),
            pl.BlockSpec((RB, F), lambda i: (i, 0)),
            pl.BlockSpec((RB, 1), lambda i: (i, 0)),
            pl.BlockSpec((1, F), lambda i: (0, 0)),
        ],
        out_specs=[
            pl.BlockSpec((RB, F), lambda i: (i, 0)),
            pl.BlockSpec((2, F), lambda i: (0, 0)),
        ],
        out_shape=[
            jax.ShapeDtypeStruct((NP, F), jnp.float32),
            jax.ShapeDtypeStruct((2, F), jnp.float32),
        ],
    )(agg, u, dinv, b)


def _out_body(agg_ref, u_ref, dinv_ref, b_ref, o_ref):
    z = (agg_ref[0] + agg_ref[1] + u_ref[...]) * dinv_ref[...] + b_ref[...]
    col = lax.broadcasted_iota(jnp.int32, (RB, 128), 1)
    zm = jnp.where(col < 40, z, -1e30)
    m = jnp.max(zm, axis=1, keepdims=True)
    s = jnp.sum(jnp.exp(zm - m), axis=1, keepdims=True)
    o_ref[...] = z - m - jnp.log(s)


def _tc_out(agg, u, dinv, b):
    return pl.pallas_call(
        _out_body,
        grid=(GRID,),
        in_specs=[
            pl.BlockSpec((2, RB, 128), lambda i: (0, i, 0)),
            pl.BlockSpec((RB, 128), lambda i: (i, 0)),
            pl.BlockSpec((RB, 1), lambda i: (i, 0)),
            pl.BlockSpec((1, 128), lambda i: (0, 0)),
        ],
        out_specs=pl.BlockSpec((RB, 128), lambda i: (i, 0)),
        out_shape=jax.ShapeDtypeStruct((NP, 128), jnp.float32),
    )(agg, u, dinv, b)


def kernel(x, edge_index, W1, b1, W2, b2, W3, b3, W4, b4,
           g1, be1, g2, be2, g3, be3):
    src = edge_index[0].astype(jnp.int32)
    dst = edge_index[1].astype(jnp.int32)
    pad = N + (jnp.arange(EP - E, dtype=jnp.int32) % (NP - N))
    src_p = jnp.concatenate([src, pad]).reshape(NW, KCH, 128)
    dst_p = jnp.concatenate([dst, pad]).reshape(NW, KCH, 128)

    x_p = jnp.pad(x, ((0, NP - N), (0, 0)))
    W3p = jnp.pad(W3, ((0, 0), (0, 64)))
    b3p = jnp.pad(b3, (0, 64))
    g3p = jnp.pad(g3, (0, 64))
    be3p = jnp.pad(be3, (0, 64))
    W4p = jnp.pad(W4, ((0, 64), (0, 88)))
    b4p = jnp.pad(b4, (0, 88))

    z128 = jnp.zeros((NP, 128), jnp.float32)

    oh = jnp.ones((128, 128), jnp.float32)
    deg_parts = _sc_degree(dst_p, z128, oh)

    b1r, b2r, b3r, b4r = (v.reshape(1, -1) for v in (b1, b2, b3p, b4p))
    g1r, g2r, g3r = (v.reshape(1, -1) for v in (g1, g2, g3p))
    be1r, be2r, be3r = (v.reshape(1, -1) for v in (be1, be2, be3p))

    dinv, u0 = _tc_dinv_scale(deg_parts, x_p)
    agg1 = _sc_scatter_128(u0, src_p, dst_p, z128)
    z1, st1 = _tc_mm1(agg1, u0, dinv, W1, b1r)

    u2 = _tc_bnmm(z1, st1, g1r, be1r, dinv, W2)
    agg2 = _sc_scatter_128(u2, src_p, dst_p, z128)
    z2, st2 = _tc_fin(agg2, u2, dinv, b2r)

    u3 = _tc_bnmm(z2, st2, g2r, be2r, dinv, W3p)
    agg3 = _sc_scatter_128(u3, src_p, dst_p, z128)
    z3, st3 = _tc_fin(agg3, u3, dinv, b3r)

    u4 = _tc_bnmm(z3, st3, g3r, be3r, dinv, W4p)
    agg4 = _sc_scatter_128(u4, src_p, dst_p, z128)
    out = _tc_out(agg4, u4, dinv, b4r)

    return out[:N, :40]
